# Initial kernel scaffold; baseline (speedup 1.0000x reference)
import jax
import jax.numpy as jnp
from jax.experimental import pallas as pl


def kernel(x, bound, xy_g0, xy_g1, xy_g2, xy_g3, yz_g0, yz_g1, yz_g2, yz_g3, xz_g0, xz_g1, xz_g2, xz_g3):
    raise NotImplementedError("write your pallas kernel here")


# speedup vs baseline: 110.7462x; 110.7462x over previous
import functools

import jax
import jax.numpy as jnp
from jax import lax
from jax.experimental import pallas as pl
from jax.experimental.pallas import tpu as pltpu
from jax.experimental.pallas import tpu_sc as plsc

N_PTS = 524288
NC, NS, LANES = 2, 16, 16
NW = NC * NS
NPW = N_PTS // NW
C = 1024
NV = C // LANES
NCHUNK = NPW // C
GSUB = 128
NSUB = C // GSUB

RES = (128, 256, 512, 1024)
PLANE_PAIRS = ((0, 1), (0, 2), (1, 2))

_OFFS = []
_off = 0
for _pi in range(3):
    for _R in RES:
        _OFFS.append(_off)
        _off += _R * _R
TBL_ROWS = _off


@functools.partial(
    pl.kernel,
    mesh=plsc.VectorSubcoreMesh(core_axis_name="c", subcore_axis_name="s"),
    out_type=jax.ShapeDtypeStruct((N_PTS, 24), jnp.float32),
    compiler_params=pltpu.CompilerParams(
        needs_layout_passes=False, use_tc_tiling_on_sc=False
    ),
    scratch_types=[
        pltpu.VMEM((3 * C,), jnp.float32),
        pltpu.VMEM((12 * C,), jnp.float32),
        pltpu.VMEM((12 * C,), jnp.int32),
        pltpu.VMEM((12 * C,), jnp.int32),
        pltpu.VMEM((C, 8), jnp.float32),
        pltpu.VMEM((C, 24), jnp.float32),
        pltpu.VMEM((2 * LANES,), jnp.float32),
        pltpu.SemaphoreType.DMA,
    ],
)
def _sc_plane_kernel(xt, tbl, par, out_hbm, xq, fr, i0r, idxr, rows, outb, parv, sem):
    wid = lax.axis_index("s") * NC + lax.axis_index("c")
    pltpu.sync_copy(par, parv)
    bv = parv[pl.ds(0, LANES)]
    inv = parv[pl.ds(LANES, LANES)]
    iota = lax.iota(jnp.int32, LANES)

    def chunk_body(ch, carry):
        base = wid * NPW + ch * C
        for a in range(3):
            pltpu.sync_copy(xt.at[pl.ds(a * N_PTS + base, C)], xq.at[pl.ds(a * C, C)])

        def coord_body(v, carry2):
            off16 = v * LANES
            for a in range(3):
                xv = xq[pl.ds(a * C + off16, LANES)]
                xn = jnp.clip((xv + bv) * inv, 0.0, 1.0)
                for l, R in enumerate(RES):
                    p = xn * (R - 1)
                    i0 = jnp.minimum(p.astype(jnp.int32), R - 2)
                    i0r[pl.ds((a * 4 + l) * C + off16, LANES)] = i0
                    fr[pl.ds((a * 4 + l) * C + off16, LANES)] = p - i0.astype(jnp.float32)
            return carry2

        lax.fori_loop(0, NV, coord_body, 0)

        def idx_body(v, carry2):
            off16 = v * LANES
            for pi, (a, b) in enumerate(PLANE_PAIRS):
                for l, R in enumerate(RES):
                    combo = pi * 4 + l
                    xi = i0r[pl.ds((a * 4 + l) * C + off16, LANES)]
                    yi = i0r[pl.ds((b * 4 + l) * C + off16, LANES)]
                    idxr[pl.ds(combo * C + off16, LANES)] = xi * R + yi + _OFFS[combo]
            return carry2

        lax.fori_loop(0, NV, idx_body, 0)

        for pi, (a, b) in enumerate(PLANE_PAIRS):
            for l in range(4):
                combo = pi * 4 + l
                copies = [
                    pltpu.async_copy(
                        tbl.at[idxr.at[pl.ds(combo * C + j * GSUB, GSUB)]],
                        rows.at[pl.ds(j * GSUB, GSUB), :],
                        sem,
                    )
                    for j in range(NSUB)
                ]
                for cp in copies:
                    cp.wait()

                fxoff = (a * 4 + l) * C
                fyoff = (b * 4 + l) * C

                def interp_body(v, carry2, fxoff=fxoff, fyoff=fyoff, combo=combo):
                    off16 = v * LANES
                    pt = iota + off16
                    fx = fr[pl.ds(fxoff + off16, LANES)]
                    fy = fr[pl.ds(fyoff + off16, LANES)]
                    g = [
                        plsc.load_gather(rows, [pt, jnp.full((LANES,), col, jnp.int32)])
                        for col in range(8)
                    ]
                    for ff in range(2):
                        a0 = g[ff] + fy * (g[2 + ff] - g[ff])
                        a1 = g[4 + ff] + fy * (g[6 + ff] - g[4 + ff])
                        o = a0 + fx * (a1 - a0)
                        plsc.store_scatter(
                            outb, [pt, jnp.full((LANES,), 2 * combo + ff, jnp.int32)], o
                        )
                    return carry2

                lax.fori_loop(0, NV, interp_body, 0)

        pltpu.sync_copy(outb, out_hbm.at[pl.ds(base, C), :])
        return carry

    lax.fori_loop(0, NCHUNK, chunk_body, 0)


def kernel(x, bound,
           xy_g0, xy_g1, xy_g2, xy_g3,
           yz_g0, yz_g1, yz_g2, yz_g3,
           xz_g0, xz_g1, xz_g2, xz_g3):
    grids = [xy_g0, xy_g1, xy_g2, xy_g3,
             yz_g0, yz_g1, yz_g2, yz_g3,
             xz_g0, xz_g1, xz_g2, xz_g3]
    quads = []
    for g in grids:
        R = g.shape[0]
        flat = g.reshape(R * R, 2)
        pair = jnp.concatenate([flat, jnp.roll(flat, -1, axis=0)], axis=1)
        quads.append(jnp.concatenate([pair, jnp.roll(pair, -R, axis=0)], axis=1))
    tbl = jnp.concatenate(quads, axis=0)
    xt = x.T.reshape(-1)
    b = jnp.asarray(bound, jnp.float32)
    par = jnp.concatenate([jnp.full((LANES,), b, jnp.float32),
                           jnp.full((LANES,), 0.5 / b, jnp.float32)])
    return _sc_plane_kernel(xt, tbl, par)
